# baseline (speedup 1.0000x reference)
import functools

import jax
import jax.numpy as jnp
from jax import lax
from jax.experimental import pallas as pl
from jax.experimental.pallas import tpu as pltpu, tpu_sc as plsc

NN = 10000
NP = 10240
EDGES = 330000
CHUNK = 2048
NCH = 164
NCHA = 166
EPA = NCHA * CHUNK
GR = CHUNK // 16
DUMMY = NN

FI = 128
HH = 8
CO1 = 128
CO2 = 64
NCLS = 16

_mesh = plsc.VectorSubcoreMesh(core_axis_name="c", subcore_axis_name="s")
_sc_params = pltpu.CompilerParams(needs_layout_passes=False)
_BLK = 512
_GRID = NP // _BLK


def _mm_kernel(w_ref, x_ref, o_ref):
    o_ref[...] = jnp.dot(w_ref[...], x_ref[...],
                         preferred_element_type=jnp.float32)


def _tc_project(wcat_t, x_t, c_out):
    fin = wcat_t.shape[1]
    return pl.pallas_call(
        _mm_kernel,
        grid=(_GRID,),
        in_specs=[
            pl.BlockSpec((c_out + 2 * HH, fin), lambda i: (0, 0)),
            pl.BlockSpec((fin, _BLK), lambda i: (0, i)),
        ],
        out_specs=pl.BlockSpec((c_out + 2 * HH, _BLK), lambda i: (0, i)),
        out_shape=jax.ShapeDtypeStruct((c_out + 2 * HH, NP), jnp.float32),
    )(wcat_t, x_t)


def _boundary_kernel(cph, num_ref, den_ref, b_ref, w_ref, o_ref):
    num = num_ref[...]
    den = den_ref[...]
    segs = []
    for h in range(HH):
        segs.append(num[h * cph:(h + 1) * cph, :] /
                    (den[h:h + 1, :] + 1e-16))
    act = jnp.concatenate(segs, axis=0)
    act = jnp.maximum(act + b_ref[...], 0.0)
    o_ref[...] = jnp.dot(w_ref[...], act, preferred_element_type=jnp.float32)


def _tc_boundary(num, den, b_col, w_t, c_in):
    cph = c_in // HH
    return pl.pallas_call(
        functools.partial(_boundary_kernel, cph),
        grid=(_GRID,),
        in_specs=[
            pl.BlockSpec((c_in, _BLK), lambda i: (0, i)),
            pl.BlockSpec((HH, _BLK), lambda i: (0, i)),
            pl.BlockSpec((c_in, 1), lambda i: (0, 0)),
            pl.BlockSpec(w_t.shape, lambda i: (0, 0)),
        ],
        out_specs=pl.BlockSpec((w_t.shape[0], _BLK), lambda i: (0, i)),
        out_shape=jax.ShapeDtypeStruct((w_t.shape[0], NP), jnp.float32),
    )(num, den, b_col, w_t)


def _gcn_boundary_kernel(num_ref, den_ref, b_ref, w_ref, cnt_ref,
                         h3_ref, dinv_ref):
    num = num_ref[...]
    den = den_ref[...]
    cph = CO2 // HH
    segs = []
    for h in range(HH):
        segs.append(num[h * cph:(h + 1) * cph, :] /
                    (den[h:h + 1, :] + 1e-16))
    act = jnp.concatenate(segs, axis=0)
    act = jnp.maximum(act + b_ref[...], 0.0)
    h3 = jnp.dot(w_ref[...], act, preferred_element_type=jnp.float32)
    cnt = cnt_ref[...]
    dinv = jnp.where(cnt > 0.0, lax.rsqrt(cnt), 0.0)
    h3_ref[...] = h3 * dinv
    dinv_ref[...] = dinv


def _tc_gcn_boundary(num, den, b_col, w_t, cnt):
    return pl.pallas_call(
        _gcn_boundary_kernel,
        grid=(_GRID,),
        in_specs=[
            pl.BlockSpec((CO2, _BLK), lambda i: (0, i)),
            pl.BlockSpec((HH, _BLK), lambda i: (0, i)),
            pl.BlockSpec((CO2, 1), lambda i: (0, 0)),
            pl.BlockSpec((NCLS, CO2), lambda i: (0, 0)),
            pl.BlockSpec((1, _BLK), lambda i: (0, i)),
        ],
        out_specs=[
            pl.BlockSpec((NCLS, _BLK), lambda i: (0, i)),
            pl.BlockSpec((1, _BLK), lambda i: (0, i)),
        ],
        out_shape=[
            jax.ShapeDtypeStruct((NCLS, NP), jnp.float32),
            jax.ShapeDtypeStruct((1, NP), jnp.float32),
        ],
    )(num, den, b_col, w_t, cnt)


def _final_kernel(part_ref, dinv_ref, bg_ref, o_ref):
    s = part_ref[0:NCLS, :] + part_ref[NCLS:2 * NCLS, :]
    s = s * dinv_ref[...]
    o_ref[...] = s.T + bg_ref[...]


def _tc_final(part, dinv, bg_row):
    return pl.pallas_call(
        _final_kernel,
        grid=(_GRID,),
        in_specs=[
            pl.BlockSpec((2 * NCLS, _BLK), lambda i: (0, i)),
            pl.BlockSpec((1, _BLK), lambda i: (0, i)),
            pl.BlockSpec((1, NCLS), lambda i: (0, 0)),
        ],
        out_specs=pl.BlockSpec((_BLK, NCLS), lambda i: (i, 0)),
        out_shape=jax.ShapeDtypeStruct((NP, NCLS), jnp.float32),
    )(part, dinv, bg_row)


def _start_chunk(eidx_hbm, ch, eb, sem):
    off = ch * 2 * CHUNK
    pltpu.make_async_copy(eidx_hbm.at[pl.ds(off, 2 * CHUNK)], eb, sem).start()


def _wait_chunk(eidx_hbm, ch, eb, sem):
    off = ch * 2 * CHUNK
    pltpu.make_async_copy(eidx_hbm.at[pl.ds(off, 2 * CHUNK)], eb, sem).wait()


def _zero_refs(refs):
    z = jnp.zeros((16,), jnp.float32)

    def body(i, carry):
        for r in refs:
            r[pl.ds(i * 16, 16)] = z
        return carry

    lax.fori_loop(0, NP // 16, body, 0)


def _make_gat_pass(c_out, cpt, with_cnt):
    n_scratch = (
        [pltpu.VMEM((NP,), jnp.float32)] * (2 * cpt + 3)
        + [pltpu.VMEM((2 * CHUNK,), jnp.int32)] * 2
        + [pltpu.SemaphoreType.DMA] * 2
    )
    outs = [
        jax.ShapeDtypeStruct((c_out, NP), jnp.float32),
        jax.ShapeDtypeStruct((HH, NP), jnp.float32),
    ]
    if with_cnt:
        outs.append(jax.ShapeDtypeStruct((NP,), jnp.float32))

    @functools.partial(
        pl.kernel,
        out_type=tuple(outs),
        mesh=_mesh,
        compiler_params=_sc_params,
        scratch_types=n_scratch,
    )
    def gat_pass(eidx_hbm, hcat_hbm, *refs):
        num_out = refs[0]
        den_out = refs[1]
        k = 3 if with_cnt else 2
        if with_cnt:
            cnt_out = refs[2]
        hcol = refs[k:k + cpt]
        accs = refs[k + cpt:k + 2 * cpt]
        asrc, adst, den_acc = refs[k + 2 * cpt:k + 2 * cpt + 3]
        eb0, eb1 = refs[k + 2 * cpt + 3:k + 2 * cpt + 5]
        sem0, sem1 = refs[k + 2 * cpt + 5:]

        wid = lax.axis_index("s") * 2 + lax.axis_index("c")
        head = wid // 4

        for j in range(cpt):
            pltpu.sync_copy(hcat_hbm.at[wid * cpt + j], hcol[j])
        pltpu.sync_copy(hcat_hbm.at[c_out + head], asrc)
        pltpu.sync_copy(hcat_hbm.at[c_out + HH + head], adst)

        _zero_refs(list(accs) + [den_acc])

        if with_cnt:
            mask_dc = jnp.broadcast_to((wid % 4 == 0) | (wid == 2), (16,))
            mask_cnt = jnp.broadcast_to(wid == 2, (16,))
            ones = jnp.ones((16,), jnp.float32)
        else:
            mask_dc = jnp.broadcast_to(wid % 4 == 0, (16,))

        def process(eb):
            @plsc.parallel_loop(0, GR, unroll=8)
            def group(g):
                s16 = eb[pl.ds(g * 16, 16)]
                d16 = eb[pl.ds(CHUNK + g * 16, 16)]
                asv = plsc.load_gather(asrc, [s16])
                adv = plsc.load_gather(adst, [d16])
                al = asv + adv
                al = jnp.where(al >= 0.0, al, 0.2 * al)
                w = jnp.exp(al)
                for j in range(cpt):
                    hv = plsc.load_gather(hcol[j], [s16])
                    plsc.addupdate_scatter(accs[j], [d16], hv * w)
                dc = jnp.where(mask_cnt, ones, w) if with_cnt else w
                plsc.addupdate_scatter(den_acc, [d16], dc, mask=mask_dc)

        _start_chunk(eidx_hbm, 0, eb0, sem0)

        def two_chunks(i, carry):
            ch0 = 2 * i
            _wait_chunk(eidx_hbm, ch0, eb0, sem0)
            _start_chunk(eidx_hbm, ch0 + 1, eb1, sem1)
            process(eb0)
            _wait_chunk(eidx_hbm, ch0 + 1, eb1, sem1)
            _start_chunk(eidx_hbm, ch0 + 2, eb0, sem0)
            process(eb1)
            return carry

        lax.fori_loop(0, NCH // 2, two_chunks, 0)
        _wait_chunk(eidx_hbm, NCH, eb0, sem0)

        for j in range(cpt):
            pltpu.sync_copy(accs[j], num_out.at[wid * cpt + j])

        @pl.when(wid % 4 == 0)
        def _():
            pltpu.sync_copy(den_acc, den_out.at[head])

        if with_cnt:
            @pl.when(wid == 2)
            def _():
                pltpu.sync_copy(den_acc, cnt_out)

    return gat_pass


_gat_pass1 = _make_gat_pass(CO1, 4, True)
_gat_pass2 = _make_gat_pass(CO2, 2, False)


@functools.partial(
    pl.kernel,
    out_type=jax.ShapeDtypeStruct((2 * NCLS, NP), jnp.float32),
    mesh=_mesh,
    compiler_params=_sc_params,
    scratch_types=(
        [pltpu.VMEM((NP,), jnp.float32)] * 2
        + [pltpu.VMEM((2 * CHUNK,), jnp.int32)] * 2
        + [pltpu.SemaphoreType.DMA] * 2
    ),
)
def _gcn_pass(eidx_hbm, h3_hbm, part_out,
              h3col, acc, eb0, eb1, sem0, sem1):
    wid = lax.axis_index("s") * 2 + lax.axis_index("c")
    col = wid % NCLS
    half = wid // NCLS
    base = half * (NCH // 2)

    pltpu.sync_copy(h3_hbm.at[col], h3col)
    _zero_refs([acc])

    def process(eb):
        @plsc.parallel_loop(0, GR, unroll=8)
        def group(g):
            s16 = eb[pl.ds(g * 16, 16)]
            d16 = eb[pl.ds(CHUNK + g * 16, 16)]
            hv = plsc.load_gather(h3col, [s16])
            plsc.addupdate_scatter(acc, [d16], hv)

    _start_chunk(eidx_hbm, base, eb0, sem0)

    def two_chunks(i, carry):
        ch0 = base + 2 * i
        _wait_chunk(eidx_hbm, ch0, eb0, sem0)
        _start_chunk(eidx_hbm, ch0 + 1, eb1, sem1)
        process(eb0)
        _wait_chunk(eidx_hbm, ch0 + 1, eb1, sem1)
        _start_chunk(eidx_hbm, ch0 + 2, eb0, sem0)
        process(eb1)
        return carry

    lax.fori_loop(0, NCH // 4, two_chunks, 0)
    _wait_chunk(eidx_hbm, base + NCH // 2, eb0, sem0)

    pltpu.sync_copy(acc, part_out.at[wid])


@jax.jit
def kernel(x, edge_index, W1, att_src1, att_dst1, b1,
           W2, att_src2, att_dst2, b2, Wg, bg):
    loop = jnp.arange(NN, dtype=jnp.int32)
    pad = jnp.full((EPA - EDGES,), DUMMY, jnp.int32)
    src = jnp.concatenate([edge_index[0], loop, pad])
    dst = jnp.concatenate([edge_index[1], loop, pad])
    eidx = jnp.stack([src.reshape(NCHA, CHUNK),
                      dst.reshape(NCHA, CHUNK)], axis=1).reshape(-1)

    x_t = jnp.pad(x, ((0, NP - NN), (0, 0))).T

    a1s = jnp.einsum("fhc,hc->fh", W1.reshape(FI, HH, -1), att_src1)
    a1d = jnp.einsum("fhc,hc->fh", W1.reshape(FI, HH, -1), att_dst1)
    wcat1_t = jnp.concatenate([W1, a1s, a1d], axis=1).T

    a2s = jnp.einsum("fhc,hc->fh", W2.reshape(CO1, HH, -1), att_src2)
    a2d = jnp.einsum("fhc,hc->fh", W2.reshape(CO1, HH, -1), att_dst2)
    wcat2_t = jnp.concatenate([W2, a2s, a2d], axis=1).T

    wg_t = Wg.T
    b1c = b1.reshape(CO1, 1)
    b2c = b2.reshape(CO2, 1)
    bg_row = bg.reshape(1, NCLS)

    hcat1 = _tc_project(wcat1_t, x_t, CO1)
    num1, den1, cnt = _gat_pass1(eidx, hcat1)

    hcat2 = _tc_boundary(num1, den1, b1c, wcat2_t, CO1)
    num2, den2 = _gat_pass2(eidx, hcat2)

    h3s, dinv = _tc_gcn_boundary(num2, den2, b2c, wg_t,
                                 cnt.reshape(1, NP))
    part = _gcn_pass(eidx, h3s)

    out = _tc_final(part, dinv, bg_row)
    return out[:NN]


# speedup vs baseline: 1.0082x; 1.0082x over previous
import functools

import jax
import jax.numpy as jnp
from jax import lax
from jax.experimental import pallas as pl
from jax.experimental.pallas import tpu as pltpu, tpu_sc as plsc

NN = 10000
NP = 10240
EDGES = 330000
CHUNK = 2048
NCH = 164
NCHA = 166
EPA = NCHA * CHUNK
GR = CHUNK // 16
DUMMY = NN

FI = 128
HH = 8
CO1 = 128
CO2 = 64
NCLS = 16

_mesh = plsc.VectorSubcoreMesh(core_axis_name="c", subcore_axis_name="s")
_sc_params = pltpu.CompilerParams(needs_layout_passes=False)
_BLK = 512
_GRID = NP // _BLK


def _mm_kernel(w_ref, x_ref, o_ref):
    o_ref[...] = jnp.dot(w_ref[...], x_ref[...],
                         preferred_element_type=jnp.float32)


def _tc_project(wcat_t, x_t, c_out):
    fin = wcat_t.shape[1]
    return pl.pallas_call(
        _mm_kernel,
        grid=(_GRID,),
        in_specs=[
            pl.BlockSpec((c_out + 2 * HH, fin), lambda i: (0, 0)),
            pl.BlockSpec((fin, _BLK), lambda i: (0, i)),
        ],
        out_specs=pl.BlockSpec((c_out + 2 * HH, _BLK), lambda i: (0, i)),
        out_shape=jax.ShapeDtypeStruct((c_out + 2 * HH, NP), jnp.float32),
    )(wcat_t, x_t)


def _boundary_kernel(cph, num_ref, den_ref, b_ref, w_ref, o_ref):
    num = num_ref[...]
    den = den_ref[...]
    segs = []
    for h in range(HH):
        segs.append(num[h * cph:(h + 1) * cph, :] /
                    (den[h:h + 1, :] + 1e-16))
    act = jnp.concatenate(segs, axis=0)
    act = jnp.maximum(act + b_ref[...], 0.0)
    o_ref[...] = jnp.dot(w_ref[...], act, preferred_element_type=jnp.float32)


def _tc_boundary(num, den, b_col, w_t, c_in):
    cph = c_in // HH
    return pl.pallas_call(
        functools.partial(_boundary_kernel, cph),
        grid=(_GRID,),
        in_specs=[
            pl.BlockSpec((c_in, _BLK), lambda i: (0, i)),
            pl.BlockSpec((HH, _BLK), lambda i: (0, i)),
            pl.BlockSpec((c_in, 1), lambda i: (0, 0)),
            pl.BlockSpec(w_t.shape, lambda i: (0, 0)),
        ],
        out_specs=pl.BlockSpec((w_t.shape[0], _BLK), lambda i: (0, i)),
        out_shape=jax.ShapeDtypeStruct((w_t.shape[0], NP), jnp.float32),
    )(num, den, b_col, w_t)


def _gcn_boundary_kernel(num_ref, den_ref, b_ref, w_ref, cnt_ref,
                         h3_ref, dinv_ref):
    num = num_ref[...]
    den = den_ref[...]
    cph = CO2 // HH
    segs = []
    for h in range(HH):
        segs.append(num[h * cph:(h + 1) * cph, :] /
                    (den[h:h + 1, :] + 1e-16))
    act = jnp.concatenate(segs, axis=0)
    act = jnp.maximum(act + b_ref[...], 0.0)
    h3 = jnp.dot(w_ref[...], act, preferred_element_type=jnp.float32)
    cnt = cnt_ref[...]
    dinv = jnp.where(cnt > 0.0, lax.rsqrt(cnt), 0.0)
    h3_ref[...] = h3 * dinv
    dinv_ref[...] = dinv


def _tc_gcn_boundary(num, den, b_col, w_t, cnt):
    return pl.pallas_call(
        _gcn_boundary_kernel,
        grid=(_GRID,),
        in_specs=[
            pl.BlockSpec((CO2, _BLK), lambda i: (0, i)),
            pl.BlockSpec((HH, _BLK), lambda i: (0, i)),
            pl.BlockSpec((CO2, 1), lambda i: (0, 0)),
            pl.BlockSpec((NCLS, CO2), lambda i: (0, 0)),
            pl.BlockSpec((1, _BLK), lambda i: (0, i)),
        ],
        out_specs=[
            pl.BlockSpec((NCLS, _BLK), lambda i: (0, i)),
            pl.BlockSpec((1, _BLK), lambda i: (0, i)),
        ],
        out_shape=[
            jax.ShapeDtypeStruct((NCLS, NP), jnp.float32),
            jax.ShapeDtypeStruct((1, NP), jnp.float32),
        ],
    )(num, den, b_col, w_t, cnt)


def _final_kernel(part_ref, dinv_ref, bg_ref, o_ref):
    s = part_ref[0:NCLS, :] + part_ref[NCLS:2 * NCLS, :]
    s = s * dinv_ref[...]
    o_ref[...] = s.T + bg_ref[...]


def _tc_final(part, dinv, bg_row):
    return pl.pallas_call(
        _final_kernel,
        grid=(_GRID,),
        in_specs=[
            pl.BlockSpec((2 * NCLS, _BLK), lambda i: (0, i)),
            pl.BlockSpec((1, _BLK), lambda i: (0, i)),
            pl.BlockSpec((1, NCLS), lambda i: (0, 0)),
        ],
        out_specs=pl.BlockSpec((_BLK, NCLS), lambda i: (i, 0)),
        out_shape=jax.ShapeDtypeStruct((NP, NCLS), jnp.float32),
    )(part, dinv, bg_row)


def _start_chunk(eidx_hbm, ch, eb, sem):
    off = ch * 2 * CHUNK
    pltpu.make_async_copy(eidx_hbm.at[pl.ds(off, 2 * CHUNK)], eb, sem).start()


def _wait_chunk(eidx_hbm, ch, eb, sem):
    off = ch * 2 * CHUNK
    pltpu.make_async_copy(eidx_hbm.at[pl.ds(off, 2 * CHUNK)], eb, sem).wait()


def _zero_refs(refs):
    z = jnp.zeros((16,), jnp.float32)

    def body(i, carry):
        for r in refs:
            r[pl.ds(i * 16, 16)] = z
        return carry

    lax.fori_loop(0, NP // 16, body, 0)


def _make_gat_pass(c_out, cpt, with_cnt):
    n_scratch = (
        [pltpu.VMEM((NP,), jnp.float32)] * (2 * cpt + 3)
        + [pltpu.VMEM((2 * CHUNK,), jnp.int32)] * 2
        + [pltpu.SemaphoreType.DMA] * 2
    )
    outs = [
        jax.ShapeDtypeStruct((c_out, NP), jnp.float32),
        jax.ShapeDtypeStruct((HH, NP), jnp.float32),
    ]
    if with_cnt:
        outs.append(jax.ShapeDtypeStruct((NP,), jnp.float32))

    @functools.partial(
        pl.kernel,
        out_type=tuple(outs),
        mesh=_mesh,
        compiler_params=_sc_params,
        scratch_types=n_scratch,
    )
    def gat_pass(eidx_hbm, hcat_hbm, *refs):
        num_out = refs[0]
        den_out = refs[1]
        k = 3 if with_cnt else 2
        if with_cnt:
            cnt_out = refs[2]
        hcol = refs[k:k + cpt]
        accs = refs[k + cpt:k + 2 * cpt]
        asrc, adst, den_acc = refs[k + 2 * cpt:k + 2 * cpt + 3]
        eb0, eb1 = refs[k + 2 * cpt + 3:k + 2 * cpt + 5]
        sem0, sem1 = refs[k + 2 * cpt + 5:]

        wid = lax.axis_index("s") * 2 + lax.axis_index("c")
        head = wid // 4

        for j in range(cpt):
            pltpu.sync_copy(hcat_hbm.at[wid * cpt + j], hcol[j])
        pltpu.sync_copy(hcat_hbm.at[c_out + head], asrc)
        pltpu.sync_copy(hcat_hbm.at[c_out + HH + head], adst)

        _zero_refs(list(accs) + [den_acc])

        own_den = ((wid % 4 == 0) & (head < 4)) | ((wid % 4 == 1) & (head >= 4))
        if with_cnt:
            mask_dc = jnp.broadcast_to(own_den | (wid == 2), (16,))
            mask_cnt = jnp.broadcast_to(wid == 2, (16,))
            ones = jnp.ones((16,), jnp.float32)
        else:
            mask_dc = jnp.broadcast_to(own_den, (16,))

        def process(eb):
            @plsc.parallel_loop(0, GR, unroll=4)
            def group(g):
                s16 = eb[pl.ds(g * 16, 16)]
                d16 = eb[pl.ds(CHUNK + g * 16, 16)]
                asv = plsc.load_gather(asrc, [s16])
                adv = plsc.load_gather(adst, [d16])
                al = asv + adv
                al = jnp.where(al >= 0.0, al, 0.2 * al)
                w = jnp.exp(al)
                for j in range(cpt):
                    hv = plsc.load_gather(hcol[j], [s16])
                    plsc.addupdate_scatter(accs[j], [d16], hv * w)
                dc = jnp.where(mask_cnt, ones, w) if with_cnt else w
                plsc.addupdate_scatter(den_acc, [d16], dc, mask=mask_dc)

        _start_chunk(eidx_hbm, 0, eb0, sem0)

        def two_chunks(i, carry):
            ch0 = 2 * i
            _wait_chunk(eidx_hbm, ch0, eb0, sem0)
            _start_chunk(eidx_hbm, ch0 + 1, eb1, sem1)
            process(eb0)
            _wait_chunk(eidx_hbm, ch0 + 1, eb1, sem1)
            _start_chunk(eidx_hbm, ch0 + 2, eb0, sem0)
            process(eb1)
            return carry

        lax.fori_loop(0, NCH // 2, two_chunks, 0)
        _wait_chunk(eidx_hbm, NCH, eb0, sem0)

        for j in range(cpt):
            pltpu.sync_copy(accs[j], num_out.at[wid * cpt + j])

        @pl.when(own_den)
        def _():
            pltpu.sync_copy(den_acc, den_out.at[head])

        if with_cnt:
            @pl.when(wid == 2)
            def _():
                pltpu.sync_copy(den_acc, cnt_out)

    return gat_pass


_gat_pass1 = _make_gat_pass(CO1, 4, True)
_gat_pass2 = _make_gat_pass(CO2, 2, False)


@functools.partial(
    pl.kernel,
    out_type=jax.ShapeDtypeStruct((2 * NCLS, NP), jnp.float32),
    mesh=_mesh,
    compiler_params=_sc_params,
    scratch_types=(
        [pltpu.VMEM((NP,), jnp.float32)] * 2
        + [pltpu.VMEM((2 * CHUNK,), jnp.int32)] * 2
        + [pltpu.SemaphoreType.DMA] * 2
    ),
)
def _gcn_pass(eidx_hbm, h3_hbm, part_out,
              h3col, acc, eb0, eb1, sem0, sem1):
    wid = lax.axis_index("s") * 2 + lax.axis_index("c")
    col = wid % NCLS
    half = wid // NCLS
    base = half * (NCH // 2)

    pltpu.sync_copy(h3_hbm.at[col], h3col)
    _zero_refs([acc])

    def process(eb):
        @plsc.parallel_loop(0, GR, unroll=4)
        def group(g):
            s16 = eb[pl.ds(g * 16, 16)]
            d16 = eb[pl.ds(CHUNK + g * 16, 16)]
            hv = plsc.load_gather(h3col, [s16])
            plsc.addupdate_scatter(acc, [d16], hv)

    _start_chunk(eidx_hbm, base, eb0, sem0)

    def two_chunks(i, carry):
        ch0 = base + 2 * i
        _wait_chunk(eidx_hbm, ch0, eb0, sem0)
        _start_chunk(eidx_hbm, ch0 + 1, eb1, sem1)
        process(eb0)
        _wait_chunk(eidx_hbm, ch0 + 1, eb1, sem1)
        _start_chunk(eidx_hbm, ch0 + 2, eb0, sem0)
        process(eb1)
        return carry

    lax.fori_loop(0, NCH // 4, two_chunks, 0)
    _wait_chunk(eidx_hbm, base + NCH // 2, eb0, sem0)

    pltpu.sync_copy(acc, part_out.at[wid])


@jax.jit
def kernel(x, edge_index, W1, att_src1, att_dst1, b1,
           W2, att_src2, att_dst2, b2, Wg, bg):
    loop = jnp.arange(NN, dtype=jnp.int32)
    pad = jnp.full((EPA - EDGES,), DUMMY, jnp.int32)
    src = jnp.concatenate([edge_index[0], loop, pad])
    dst = jnp.concatenate([edge_index[1], loop, pad])
    eidx = jnp.stack([src.reshape(NCHA, CHUNK),
                      dst.reshape(NCHA, CHUNK)], axis=1).reshape(-1)

    x_t = jnp.pad(x, ((0, NP - NN), (0, 0))).T

    a1s = jnp.einsum("fhc,hc->fh", W1.reshape(FI, HH, -1), att_src1)
    a1d = jnp.einsum("fhc,hc->fh", W1.reshape(FI, HH, -1), att_dst1)
    wcat1_t = jnp.concatenate([W1, a1s, a1d], axis=1).T

    a2s = jnp.einsum("fhc,hc->fh", W2.reshape(CO1, HH, -1), att_src2)
    a2d = jnp.einsum("fhc,hc->fh", W2.reshape(CO1, HH, -1), att_dst2)
    wcat2_t = jnp.concatenate([W2, a2s, a2d], axis=1).T

    wg_t = Wg.T
    b1c = b1.reshape(CO1, 1)
    b2c = b2.reshape(CO2, 1)
    bg_row = bg.reshape(1, NCLS)

    hcat1 = _tc_project(wcat1_t, x_t, CO1)
    num1, den1, cnt = _gat_pass1(eidx, hcat1)

    hcat2 = _tc_boundary(num1, den1, b1c, wcat2_t, CO1)
    num2, den2 = _gat_pass2(eidx, hcat2)

    h3s, dinv = _tc_gcn_boundary(num2, den2, b2c, wg_t,
                                 cnt.reshape(1, NP))
    part = _gcn_pass(eidx, h3s)

    out = _tc_final(part, dinv, bg_row)
    return out[:NN]


# speedup vs baseline: 1.0647x; 1.0560x over previous
import functools

import jax
import jax.numpy as jnp
from jax import lax
from jax.experimental import pallas as pl
from jax.experimental.pallas import tpu as pltpu, tpu_sc as plsc

NN = 10000
NP = 10240
EDGES = 330000
CHUNK = 2048
NCH = 164
NCHA = 166
EPA = NCHA * CHUNK
GR = CHUNK // 16
DUMMY = NN

FI = 128
HH = 8
CO1 = 128
CO2 = 64
NCLS = 16

_mesh = plsc.VectorSubcoreMesh(core_axis_name="c", subcore_axis_name="s")
_sc_params = pltpu.CompilerParams(needs_layout_passes=False)
_BLK = NP
_GRID = NP // _BLK


def _mm_kernel(w_ref, x_ref, o_ref):
    o_ref[...] = jnp.dot(w_ref[...], x_ref[...],
                         preferred_element_type=jnp.float32)


def _tc_project(wcat_t, x_t, c_out):
    fin = wcat_t.shape[1]
    return pl.pallas_call(
        _mm_kernel,
        grid=(_GRID,),
        in_specs=[
            pl.BlockSpec((c_out + 2 * HH, fin), lambda i: (0, 0)),
            pl.BlockSpec((fin, _BLK), lambda i: (0, i)),
        ],
        out_specs=pl.BlockSpec((c_out + 2 * HH, _BLK), lambda i: (0, i)),
        out_shape=jax.ShapeDtypeStruct((c_out + 2 * HH, NP), jnp.float32),
    )(wcat_t, x_t)


def _boundary_kernel(cph, num_ref, den_ref, b_ref, w_ref, o_ref):
    num = num_ref[...]
    den = den_ref[...]
    segs = []
    for h in range(HH):
        segs.append(num[h * cph:(h + 1) * cph, :] /
                    (den[h:h + 1, :] + 1e-16))
    act = jnp.concatenate(segs, axis=0)
    act = jnp.maximum(act + b_ref[...], 0.0)
    o_ref[...] = jnp.dot(w_ref[...], act, preferred_element_type=jnp.float32)


def _tc_boundary(num, den, b_col, w_t, c_in):
    cph = c_in // HH
    return pl.pallas_call(
        functools.partial(_boundary_kernel, cph),
        grid=(_GRID,),
        in_specs=[
            pl.BlockSpec((c_in, _BLK), lambda i: (0, i)),
            pl.BlockSpec((HH, _BLK), lambda i: (0, i)),
            pl.BlockSpec((c_in, 1), lambda i: (0, 0)),
            pl.BlockSpec(w_t.shape, lambda i: (0, 0)),
        ],
        out_specs=pl.BlockSpec((w_t.shape[0], _BLK), lambda i: (0, i)),
        out_shape=jax.ShapeDtypeStruct((w_t.shape[0], NP), jnp.float32),
    )(num, den, b_col, w_t)


def _gcn_boundary_kernel(num_ref, den_ref, b_ref, w_ref, cnt_ref,
                         h3_ref, dinv_ref):
    num = num_ref[...]
    den = den_ref[...]
    cph = CO2 // HH
    segs = []
    for h in range(HH):
        segs.append(num[h * cph:(h + 1) * cph, :] /
                    (den[h:h + 1, :] + 1e-16))
    act = jnp.concatenate(segs, axis=0)
    act = jnp.maximum(act + b_ref[...], 0.0)
    h3 = jnp.dot(w_ref[...], act, preferred_element_type=jnp.float32)
    cnt = cnt_ref[...]
    dinv = jnp.where(cnt > 0.0, lax.rsqrt(cnt), 0.0)
    h3_ref[...] = h3 * dinv
    dinv_ref[...] = dinv


def _tc_gcn_boundary(num, den, b_col, w_t, cnt):
    return pl.pallas_call(
        _gcn_boundary_kernel,
        grid=(_GRID,),
        in_specs=[
            pl.BlockSpec((CO2, _BLK), lambda i: (0, i)),
            pl.BlockSpec((HH, _BLK), lambda i: (0, i)),
            pl.BlockSpec((CO2, 1), lambda i: (0, 0)),
            pl.BlockSpec((NCLS, CO2), lambda i: (0, 0)),
            pl.BlockSpec((1, _BLK), lambda i: (0, i)),
        ],
        out_specs=[
            pl.BlockSpec((NCLS, _BLK), lambda i: (0, i)),
            pl.BlockSpec((1, _BLK), lambda i: (0, i)),
        ],
        out_shape=[
            jax.ShapeDtypeStruct((NCLS, NP), jnp.float32),
            jax.ShapeDtypeStruct((1, NP), jnp.float32),
        ],
    )(num, den, b_col, w_t, cnt)


def _final_kernel(part_ref, dinv_ref, bg_ref, o_ref):
    s = part_ref[0:NCLS, :] + part_ref[NCLS:2 * NCLS, :]
    s = s * dinv_ref[...]
    o_ref[...] = s.T + bg_ref[...]


def _tc_final(part, dinv, bg_row):
    return pl.pallas_call(
        _final_kernel,
        grid=(_GRID,),
        in_specs=[
            pl.BlockSpec((2 * NCLS, _BLK), lambda i: (0, i)),
            pl.BlockSpec((1, _BLK), lambda i: (0, i)),
            pl.BlockSpec((1, NCLS), lambda i: (0, 0)),
        ],
        out_specs=pl.BlockSpec((_BLK, NCLS), lambda i: (i, 0)),
        out_shape=jax.ShapeDtypeStruct((NP, NCLS), jnp.float32),
    )(part, dinv, bg_row)


def _start_chunk(eidx_hbm, ch, eb, sem):
    off = ch * 2 * CHUNK
    pltpu.make_async_copy(eidx_hbm.at[pl.ds(off, 2 * CHUNK)], eb, sem).start()


def _wait_chunk(eidx_hbm, ch, eb, sem):
    off = ch * 2 * CHUNK
    pltpu.make_async_copy(eidx_hbm.at[pl.ds(off, 2 * CHUNK)], eb, sem).wait()


def _zero_refs(refs):
    z = jnp.zeros((16,), jnp.float32)

    def body(i, carry):
        for r in refs:
            r[pl.ds(i * 16, 16)] = z
        return carry

    lax.fori_loop(0, NP // 16, body, 0)


def _make_gat_pass(c_out, cpt, with_cnt):
    n_scratch = (
        [pltpu.VMEM((NP,), jnp.float32)] * (2 * cpt + 3)
        + [pltpu.VMEM((2 * CHUNK,), jnp.int32)] * 2
        + [pltpu.SemaphoreType.DMA] * 2
    )
    outs = [
        jax.ShapeDtypeStruct((c_out, NP), jnp.float32),
        jax.ShapeDtypeStruct((HH, NP), jnp.float32),
    ]
    if with_cnt:
        outs.append(jax.ShapeDtypeStruct((NP,), jnp.float32))

    @functools.partial(
        pl.kernel,
        out_type=tuple(outs),
        mesh=_mesh,
        compiler_params=_sc_params,
        scratch_types=n_scratch,
    )
    def gat_pass(eidx_hbm, hcat_hbm, *refs):
        num_out = refs[0]
        den_out = refs[1]
        k = 3 if with_cnt else 2
        if with_cnt:
            cnt_out = refs[2]
        hcol = refs[k:k + cpt]
        accs = refs[k + cpt:k + 2 * cpt]
        asrc, adst, den_acc = refs[k + 2 * cpt:k + 2 * cpt + 3]
        eb0, eb1 = refs[k + 2 * cpt + 3:k + 2 * cpt + 5]
        sem0, sem1 = refs[k + 2 * cpt + 5:]

        wid = lax.axis_index("s") * 2 + lax.axis_index("c")
        head = wid // 4

        for j in range(cpt):
            pltpu.sync_copy(hcat_hbm.at[wid * cpt + j], hcol[j])
        pltpu.sync_copy(hcat_hbm.at[c_out + head], asrc)
        pltpu.sync_copy(hcat_hbm.at[c_out + HH + head], adst)

        _zero_refs(list(accs) + [den_acc])

        own_den = ((wid % 4 == 0) & (head < 4)) | ((wid % 4 == 1) & (head >= 4))
        if with_cnt:
            mask_dc = jnp.broadcast_to(own_den | (wid == 2), (16,))
            mask_cnt = jnp.broadcast_to(wid == 2, (16,))
            ones = jnp.ones((16,), jnp.float32)
        else:
            mask_dc = jnp.broadcast_to(own_den, (16,))

        def process(eb):
            @plsc.parallel_loop(0, GR, unroll=4)
            def group(g):
                s16 = eb[pl.ds(g * 16, 16)]
                d16 = eb[pl.ds(CHUNK + g * 16, 16)]
                asv = plsc.load_gather(asrc, [s16])
                adv = plsc.load_gather(adst, [d16])
                al = asv + adv
                al = jnp.where(al >= 0.0, al, 0.2 * al)
                w = jnp.exp(al)
                for j in range(cpt):
                    hv = plsc.load_gather(hcol[j], [s16])
                    plsc.addupdate_scatter(accs[j], [d16], hv * w)
                dc = jnp.where(mask_cnt, ones, w) if with_cnt else w
                plsc.addupdate_scatter(den_acc, [d16], dc, mask=mask_dc)

        _start_chunk(eidx_hbm, 0, eb0, sem0)

        def two_chunks(i, carry):
            ch0 = 2 * i
            _wait_chunk(eidx_hbm, ch0, eb0, sem0)
            _start_chunk(eidx_hbm, ch0 + 1, eb1, sem1)
            process(eb0)
            _wait_chunk(eidx_hbm, ch0 + 1, eb1, sem1)
            _start_chunk(eidx_hbm, ch0 + 2, eb0, sem0)
            process(eb1)
            return carry

        lax.fori_loop(0, NCH // 2, two_chunks, 0)
        _wait_chunk(eidx_hbm, NCH, eb0, sem0)

        for j in range(cpt):
            pltpu.sync_copy(accs[j], num_out.at[wid * cpt + j])

        @pl.when(own_den)
        def _():
            pltpu.sync_copy(den_acc, den_out.at[head])

        if with_cnt:
            @pl.when(wid == 2)
            def _():
                pltpu.sync_copy(den_acc, cnt_out)

    return gat_pass


_gat_pass1 = _make_gat_pass(CO1, 4, True)
_gat_pass2 = _make_gat_pass(CO2, 2, False)


@functools.partial(
    pl.kernel,
    out_type=jax.ShapeDtypeStruct((2 * NCLS, NP), jnp.float32),
    mesh=_mesh,
    compiler_params=_sc_params,
    scratch_types=(
        [pltpu.VMEM((NP,), jnp.float32)] * 2
        + [pltpu.VMEM((2 * CHUNK,), jnp.int32)] * 2
        + [pltpu.SemaphoreType.DMA] * 2
    ),
)
def _gcn_pass(eidx_hbm, h3_hbm, part_out,
              h3col, acc, eb0, eb1, sem0, sem1):
    wid = lax.axis_index("s") * 2 + lax.axis_index("c")
    col = wid % NCLS
    half = wid // NCLS
    base = half * (NCH // 2)

    pltpu.sync_copy(h3_hbm.at[col], h3col)
    _zero_refs([acc])

    def process(eb):
        @plsc.parallel_loop(0, GR, unroll=4)
        def group(g):
            s16 = eb[pl.ds(g * 16, 16)]
            d16 = eb[pl.ds(CHUNK + g * 16, 16)]
            hv = plsc.load_gather(h3col, [s16])
            plsc.addupdate_scatter(acc, [d16], hv)

    _start_chunk(eidx_hbm, base, eb0, sem0)

    def two_chunks(i, carry):
        ch0 = base + 2 * i
        _wait_chunk(eidx_hbm, ch0, eb0, sem0)
        _start_chunk(eidx_hbm, ch0 + 1, eb1, sem1)
        process(eb0)
        _wait_chunk(eidx_hbm, ch0 + 1, eb1, sem1)
        _start_chunk(eidx_hbm, ch0 + 2, eb0, sem0)
        process(eb1)
        return carry

    lax.fori_loop(0, NCH // 4, two_chunks, 0)
    _wait_chunk(eidx_hbm, base + NCH // 2, eb0, sem0)

    pltpu.sync_copy(acc, part_out.at[wid])


@jax.jit
def kernel(x, edge_index, W1, att_src1, att_dst1, b1,
           W2, att_src2, att_dst2, b2, Wg, bg):
    loop = jnp.arange(NN, dtype=jnp.int32)
    pad = jnp.full((EPA - EDGES,), DUMMY, jnp.int32)
    src = jnp.concatenate([edge_index[0], loop, pad])
    dst = jnp.concatenate([edge_index[1], loop, pad])
    eidx = jnp.stack([src.reshape(NCHA, CHUNK),
                      dst.reshape(NCHA, CHUNK)], axis=1).reshape(-1)

    x_t = jnp.pad(x, ((0, NP - NN), (0, 0))).T

    def _fold(att, co):
        sel = (jnp.arange(co)[:, None] // (co // HH)
               == jnp.arange(HH)[None, :]).astype(jnp.float32)
        return att.reshape(co, 1) * sel

    wcat1_t = jnp.concatenate(
        [W1, W1 @ _fold(att_src1, CO1), W1 @ _fold(att_dst1, CO1)],
        axis=1).T
    wcat2_t = jnp.concatenate(
        [W2, W2 @ _fold(att_src2, CO2), W2 @ _fold(att_dst2, CO2)],
        axis=1).T

    wg_t = Wg.T
    b1c = b1.reshape(CO1, 1)
    b2c = b2.reshape(CO2, 1)
    bg_row = bg.reshape(1, NCLS)

    hcat1 = _tc_project(wcat1_t, x_t, CO1)
    num1, den1, cnt = _gat_pass1(eidx, hcat1)

    hcat2 = _tc_boundary(num1, den1, b1c, wcat2_t, CO1)
    num2, den2 = _gat_pass2(eidx, hcat2)

    h3s, dinv = _tc_gcn_boundary(num2, den2, b2c, wg_t,
                                 cnt.reshape(1, NP))
    part = _gcn_pass(eidx, h3s)

    out = _tc_final(part, dinv, bg_row)
    return out[:NN]
